# baseline (speedup 1.0000x reference)
import functools
import math

import jax
import jax.numpy as jnp
from jax import lax
from jax.experimental import pallas as pl
from jax.experimental.pallas import tpu as pltpu
from jax.experimental.pallas import tpu_sc as plsc

VOCAB_N = 100000
DMODEL = 512
SCALE = float(math.sqrt(DMODEL))

NUM_CORES = 2
NUM_SUBCORES = 16
NW = NUM_CORES * NUM_SUBCORES

B_TOTAL = 4096 * 50
B_PER_W = B_TOTAL // NW
CHUNK = 64
NCHUNK = B_PER_W // CHUNK
LANES = 16
VECS_PER_ROW = DMODEL // LANES


def _body(x_hbm, table_hbm, out_hbm, idx_v, buf, sem):
    wid = lax.axis_index("s") * NUM_CORES + lax.axis_index("c")
    pltpu.sync_copy(x_hbm.at[wid], idx_v)
    row_base = wid * B_PER_W

    def chunk_body(g, carry):
        pltpu.async_copy(table_hbm.at[idx_v.at[g]], buf, sem).wait()

        def row_body(r, c2):
            for c in range(VECS_PER_ROW):
                sl = (r, pl.ds(c * LANES, LANES))
                buf[sl] = buf[sl] * SCALE
            return c2

        lax.fori_loop(0, CHUNK, row_body, 0)
        pltpu.sync_copy(buf, out_hbm.at[pl.ds(row_base + g * CHUNK, CHUNK)])
        return carry

    lax.fori_loop(0, NCHUNK, chunk_body, 0)


@jax.jit
def _lookup(xf, table):
    mesh = plsc.VectorSubcoreMesh(core_axis_name="c", subcore_axis_name="s")
    k = functools.partial(
        pl.kernel,
        mesh=mesh,
        out_type=jax.ShapeDtypeStruct((B_TOTAL, DMODEL), jnp.float32),
        scratch_types=[
            pltpu.VMEM((NCHUNK, CHUNK), jnp.int32),
            pltpu.VMEM((CHUNK, DMODEL), jnp.float32),
            pltpu.SemaphoreType.DMA,
        ],
    )(_body)
    return k(xf, table)


def kernel(x, table):
    xf = x.reshape(NW, NCHUNK, CHUNK).astype(jnp.int32)
    out = _lookup(xf, table)
    return out.reshape(x.shape[0], x.shape[1], DMODEL)


# speedup vs baseline: 1.1775x; 1.1775x over previous
import functools
import math

import jax
import jax.numpy as jnp
from jax import lax
from jax.experimental import pallas as pl
from jax.experimental.pallas import tpu as pltpu
from jax.experimental.pallas import tpu_sc as plsc

VOCAB_N = 100000
DMODEL = 512
SCALE = float(math.sqrt(DMODEL))

NUM_CORES = 2
NUM_SUBCORES = 16
NW = NUM_CORES * NUM_SUBCORES

B_TOTAL = 4096 * 50
B_PER_W = B_TOTAL // NW
CHUNK = 40
NCHUNK = B_PER_W // CHUNK
LANES = 16
VECS_PER_ROW = DMODEL // LANES


def _body(x_hbm, table_hbm, out_hbm, idx_v, gin0, gin1, gout0, gout1,
          gs0, gs1, os0, os1):
    wid = lax.axis_index("s") * NUM_CORES + lax.axis_index("c")
    pltpu.sync_copy(x_hbm.at[wid], idx_v)
    row_base = wid * B_PER_W
    gin = (gin0, gin1)
    gout = (gout0, gout1)
    gsem = (gs0, gs1)
    osem = (os0, os1)

    def g_src(c):
        return table_hbm.at[idx_v.at[c]]

    def o_dst(c):
        return out_hbm.at[pl.ds(row_base + c * CHUNK, CHUNK)]

    pltpu.async_copy(g_src(0), gin[0], gsem[0])

    def outer(g2, carry):
        for b in range(2):
            c = 2 * g2 + b

            @pl.when(c + 1 < NCHUNK)
            def _start_next():
                pltpu.async_copy(g_src(c + 1), gin[1 - b], gsem[1 - b])

            pltpu.make_async_copy(g_src(c), gin[b], gsem[b]).wait()

            @pl.when(c >= 2)
            def _drain_out():
                pltpu.make_async_copy(gout[b], o_dst(c - 2), osem[b]).wait()

            def row_body(r, acc):
                for v in range(VECS_PER_ROW):
                    sl = (r, pl.ds(v * LANES, LANES))
                    gout[b][sl] = gin[b][sl] * SCALE
                return acc

            lax.fori_loop(0, CHUNK, row_body, 0)
            pltpu.async_copy(gout[b], o_dst(c), osem[b])
        return carry

    lax.fori_loop(0, NCHUNK // 2, outer, 0)
    pltpu.make_async_copy(gout[0], o_dst(NCHUNK - 2), osem[0]).wait()
    pltpu.make_async_copy(gout[1], o_dst(NCHUNK - 1), osem[1]).wait()


@jax.jit
def _lookup(xf, table):
    mesh = plsc.VectorSubcoreMesh(core_axis_name="c", subcore_axis_name="s")
    k = functools.partial(
        pl.kernel,
        mesh=mesh,
        out_type=jax.ShapeDtypeStruct((B_TOTAL, DMODEL), jnp.float32),
        scratch_types=[
            pltpu.VMEM((NCHUNK, CHUNK), jnp.int32),
            pltpu.VMEM((CHUNK, DMODEL), jnp.float32),
            pltpu.VMEM((CHUNK, DMODEL), jnp.float32),
            pltpu.VMEM((CHUNK, DMODEL), jnp.float32),
            pltpu.VMEM((CHUNK, DMODEL), jnp.float32),
            pltpu.SemaphoreType.DMA,
            pltpu.SemaphoreType.DMA,
            pltpu.SemaphoreType.DMA,
            pltpu.SemaphoreType.DMA,
        ],
    )(_body)
    return k(xf, table)


def kernel(x, table):
    xf = x.reshape(NW, NCHUNK, CHUNK).astype(jnp.int32)
    out = _lookup(xf, table)
    return out.reshape(x.shape[0], x.shape[1], DMODEL)


# speedup vs baseline: 1.7349x; 1.4733x over previous
import functools
import math

import jax
import jax.numpy as jnp
from jax import lax
from jax.experimental import pallas as pl
from jax.experimental.pallas import tpu as pltpu
from jax.experimental.pallas import tpu_sc as plsc

VOCAB_N = 100000
DMODEL = 512
SCALE = float(math.sqrt(DMODEL))

NUM_CORES = 2
NUM_SUBCORES = 16
NW = NUM_CORES * NUM_SUBCORES

BATCH = 4096
SEQ = 50
ROWS_PER_W = BATCH // NW
LANES = 16
VECS_PER_ROW = DMODEL // LANES


def _body(x_hbm, table_hbm, out_hbm, idx_v, gin0, gin1, gout0, gout1,
          gs0, gs1, os0, os1):
    wid = lax.axis_index("s") * NUM_CORES + lax.axis_index("c")
    row_base = wid * ROWS_PER_W
    pltpu.sync_copy(x_hbm.at[pl.ds(row_base, ROWS_PER_W)], idx_v)
    gin = (gin0, gin1)
    gout = (gout0, gout1)
    gsem = (gs0, gs1)
    osem = (os0, os1)

    def g_src(c):
        return table_hbm.at[idx_v.at[c]]

    def o_dst(c):
        return out_hbm.at[row_base + c]

    pltpu.async_copy(g_src(0), gin[0], gsem[0])

    def outer(g2, carry):
        for b in range(2):
            c = 2 * g2 + b

            @pl.when(c + 1 < ROWS_PER_W)
            def _start_next():
                pltpu.async_copy(g_src(c + 1), gin[1 - b], gsem[1 - b])

            pltpu.make_async_copy(g_src(c), gin[b], gsem[b]).wait()

            @pl.when(c >= 2)
            def _drain_out():
                pltpu.make_async_copy(gout[b], o_dst(c - 2), osem[b]).wait()

            def row_body(r, acc):
                for v in range(VECS_PER_ROW):
                    sl = (r, pl.ds(v * LANES, LANES))
                    gout[b][sl] = gin[b][sl] * SCALE
                return acc

            lax.fori_loop(0, SEQ, row_body, 0)
            pltpu.async_copy(gout[b], o_dst(c), osem[b])
        return carry

    lax.fori_loop(0, ROWS_PER_W // 2, outer, 0)
    pltpu.make_async_copy(gout[0], o_dst(ROWS_PER_W - 2), osem[0]).wait()
    pltpu.make_async_copy(gout[1], o_dst(ROWS_PER_W - 1), osem[1]).wait()


@jax.jit
def _lookup(xf, table):
    mesh = plsc.VectorSubcoreMesh(core_axis_name="c", subcore_axis_name="s")
    k = functools.partial(
        pl.kernel,
        mesh=mesh,
        out_type=jax.ShapeDtypeStruct((BATCH, SEQ, DMODEL), jnp.float32),
        scratch_types=[
            pltpu.VMEM((ROWS_PER_W, SEQ), jnp.int32),
            pltpu.VMEM((SEQ, DMODEL), jnp.float32),
            pltpu.VMEM((SEQ, DMODEL), jnp.float32),
            pltpu.VMEM((SEQ, DMODEL), jnp.float32),
            pltpu.VMEM((SEQ, DMODEL), jnp.float32),
            pltpu.SemaphoreType.DMA,
            pltpu.SemaphoreType.DMA,
            pltpu.SemaphoreType.DMA,
            pltpu.SemaphoreType.DMA,
        ],
    )(_body)
    return k(xf, table)


def kernel(x, table):
    return _lookup(x.astype(jnp.int32), table)


# speedup vs baseline: 3.6563x; 2.1075x over previous
import functools
import math

import jax
import jax.numpy as jnp
from jax import lax
from jax.experimental import pallas as pl
from jax.experimental.pallas import tpu as pltpu
from jax.experimental.pallas import tpu_sc as plsc

VOCAB_N = 100000
DMODEL = 512
SCALE = float(math.sqrt(DMODEL))

NUM_CORES = 2
NUM_SUBCORES = 16
NW = NUM_CORES * NUM_SUBCORES

B_TOTAL = 4096 * 50
B_PER_W = B_TOTAL // NW
CHUNK = 40
NCHUNK = B_PER_W // CHUNK
LANES = 16
VECS_PER_ROW = DMODEL // LANES


def _body(x_hbm, table_hbm, out_hbm, idx_v, gin0, gin1, gout0, gout1,
          gs0, gs1, os0, os1):
    wid = lax.axis_index("s") * NUM_CORES + lax.axis_index("c")
    pltpu.sync_copy(x_hbm.at[wid], idx_v)
    row_base = wid * B_PER_W
    gin = (gin0, gin1)
    gout = (gout0, gout1)
    gsem = (gs0, gs1)
    osem = (os0, os1)

    def g_src(c):
        return table_hbm.at[idx_v.at[c]]

    def o_dst(c):
        return out_hbm.at[pl.ds(row_base + c * CHUNK, CHUNK)]

    pltpu.async_copy(g_src(0), gin[0], gsem[0])

    def outer(g2, carry):
        for b in range(2):
            c = 2 * g2 + b

            @pl.when(c + 1 < NCHUNK)
            def _start_next():
                pltpu.async_copy(g_src(c + 1), gin[1 - b], gsem[1 - b])

            pltpu.make_async_copy(g_src(c), gin[b], gsem[b]).wait()

            @pl.when(c >= 2)
            def _drain_out():
                pltpu.make_async_copy(gout[b], o_dst(c - 2), osem[b]).wait()

            def row_body(r, acc):
                for v in range(VECS_PER_ROW):
                    sl = (r, pl.ds(v * LANES, LANES))
                    gout[b][sl] = gin[b][sl] * SCALE
                return acc

            lax.fori_loop(0, CHUNK, row_body, 0)
            pltpu.async_copy(gout[b], o_dst(c), osem[b])
        return carry

    lax.fori_loop(0, NCHUNK // 2, outer, 0)
    pltpu.make_async_copy(gout[0], o_dst(NCHUNK - 2), osem[0]).wait()
    pltpu.make_async_copy(gout[1], o_dst(NCHUNK - 1), osem[1]).wait()


@jax.jit
def _lookup(xf, table):
    mesh = plsc.VectorSubcoreMesh(core_axis_name="c", subcore_axis_name="s")
    k = functools.partial(
        pl.kernel,
        mesh=mesh,
        out_type=jax.ShapeDtypeStruct((B_TOTAL, DMODEL), jnp.float32),
        scratch_types=[
            pltpu.VMEM((NCHUNK, CHUNK), jnp.int32),
            pltpu.VMEM((CHUNK, DMODEL), jnp.float32),
            pltpu.VMEM((CHUNK, DMODEL), jnp.float32),
            pltpu.VMEM((CHUNK, DMODEL), jnp.float32),
            pltpu.VMEM((CHUNK, DMODEL), jnp.float32),
            pltpu.SemaphoreType.DMA,
            pltpu.SemaphoreType.DMA,
            pltpu.SemaphoreType.DMA,
            pltpu.SemaphoreType.DMA,
        ],
    )(_body)
    return k(xf, table)


def kernel(x, table):
    batch, seq = x.shape
    xf = x.T.reshape(NW, NCHUNK, CHUNK).astype(jnp.int32)
    out = _lookup(xf, table)
    return out.reshape(seq, batch, DMODEL).transpose(1, 0, 2)
